# baseline (speedup 1.0000x reference)
import jax
import jax.numpy as jnp
from jax import lax
from jax.experimental import pallas as pl
from jax.experimental.pallas import tpu as pltpu

B, C, N = 8, 1024, 4096
J, R = 64, 32
NBLK = 512
NBLKS = N // NBLK
NT = 32


def _bitonic_sort_desc(arr):
    n = len(arr)
    k = 2
    while k <= n:
        j = k // 2
        while j >= 1:
            for i in range(n):
                p = i ^ j
                if p > i:
                    a, b = arr[i], arr[p]
                    hi, lo = jnp.maximum(a, b), jnp.minimum(a, b)
                    if (i & k) == 0:
                        arr[i], arr[p] = hi, lo
                    else:
                        arr[i], arr[p] = lo, hi
            j //= 2
        k *= 2
    return arr


def _bitonic_clean_desc(arr):
    n = len(arr)
    j = n // 2
    while j >= 1:
        for i in range(n):
            p = i ^ j
            if p > i:
                a, b = arr[i], arr[p]
                arr[i], arr[p] = jnp.maximum(a, b), jnp.minimum(a, b)
        j //= 2
    return arr


def _conv_topk_body(x_ref, w_ref, b_ref, out_ref, y_sc):
    n = pl.program_id(1)
    yb = jnp.dot(w_ref[...], x_ref[0], preferred_element_type=jnp.float32)
    y_sc[:, pl.ds(n * NBLK, NBLK)] = yb + b_ref[...]

    @pl.when(n == NBLKS - 1)
    def _():
        tiles = [y_sc[:, a * 128:(a + 1) * 128] for a in range(NT)]
        tiles = _bitonic_sort_desc(tiles)

        w = 64
        packed = []
        for a in range(NT):
            lo_a, hi_a = tiles[a][:, :w], tiles[NT - 1 - a][:, w:]
            packed.append(jnp.concatenate(
                [jnp.maximum(lo_a, hi_a), jnp.minimum(lo_a, hi_a)], axis=1))
        packed = _bitonic_clean_desc(packed)

        while w > 1:
            h = w // 2
            nxt = []
            for a in range(NT):
                am, bm = packed[a][:, :h], packed[NT - 1 - a][:, h:w]
                an, bn = packed[a][:, w:w + h], packed[NT - 1 - a][:, w + h:]
                nxt.append(jnp.concatenate(
                    [jnp.maximum(am, bm), jnp.minimum(an, bn)], axis=1))
            packed = _bitonic_clean_desc(nxt)
            w = h

        cols = ([packed[a][:, 0:1] for a in range(NT)]
                + [packed[a][:, 1:2] for a in range(NT)])
        out_ref[0] = jnp.concatenate(cols, axis=1)


def _mlp_body(f_ref, w1_ref, b1_ref, w2_ref, b2_ref, w3_ref, b3_ref, out_ref):
    h = jax.nn.sigmoid(
        jnp.dot(f_ref[...], w1_ref[...], preferred_element_type=jnp.float32)
        + b1_ref[...]
    )
    h = jax.nn.sigmoid(
        jnp.dot(h, w2_ref[...], preferred_element_type=jnp.float32) + b2_ref[...]
    )
    out_ref[...] = (
        jnp.dot(h, w3_ref[...], preferred_element_type=jnp.float32) + b3_ref[...]
    )


@jax.jit
def kernel(x, conv_w, conv_b, w1, b1, w2, b2, w3, b3):
    topk = pl.pallas_call(
        _conv_topk_body,
        grid=(B, NBLKS),
        in_specs=[
            pl.BlockSpec((1, C, NBLK), lambda b, n: (b, 0, n)),
            pl.BlockSpec((J, C), lambda b, n: (0, 0)),
            pl.BlockSpec((J, 1), lambda b, n: (0, 0)),
        ],
        out_specs=pl.BlockSpec((1, J, 2 * R), lambda b, n: (b, 0, 0)),
        out_shape=jax.ShapeDtypeStruct((B, J, 2 * R), jnp.float32),
        scratch_shapes=[pltpu.VMEM((J, N), jnp.float32)],
        compiler_params=pltpu.CompilerParams(
            dimension_semantics=("arbitrary", "arbitrary"),
        ),
    )(x, conv_w, conv_b.reshape(J, 1))

    flat = topk.reshape(B, 2 * R * J)
    logits = pl.pallas_call(
        _mlp_body,
        in_specs=[pl.BlockSpec(a.shape, lambda: (0,) * a.ndim) for a in
                  (flat, w1, b1.reshape(1, -1), w2, b2.reshape(1, -1),
                   w3, b3.reshape(1, -1))],
        out_specs=pl.BlockSpec((B, 2), lambda: (0, 0)),
        out_shape=jax.ShapeDtypeStruct((B, 2), jnp.float32),
    )(flat, w1, b1.reshape(1, -1), w2, b2.reshape(1, -1), w3, b3.reshape(1, -1))
    return logits


# speedup vs baseline: 2.0906x; 2.0906x over previous
import jax
import jax.numpy as jnp
from jax import lax
from jax.experimental import pallas as pl
from jax.experimental.pallas import tpu as pltpu

B, C, N = 8, 1024, 4096
J, R = 64, 32
NBLK = 512
NBLKS = N // NBLK
NT = 32


def _bitonic_sort_desc(arr):
    n = len(arr)
    k = 2
    while k <= n:
        j = k // 2
        while j >= 1:
            for i in range(n):
                p = i ^ j
                if p > i:
                    a, b = arr[i], arr[p]
                    hi, lo = jnp.maximum(a, b), jnp.minimum(a, b)
                    if (i & k) == 0:
                        arr[i], arr[p] = hi, lo
                    else:
                        arr[i], arr[p] = lo, hi
            j //= 2
        k *= 2
    return arr


def _bitonic_clean_desc(arr):
    n = len(arr)
    j = n // 2
    while j >= 1:
        for i in range(n):
            p = i ^ j
            if p > i:
                a, b = arr[i], arr[p]
                arr[i], arr[p] = jnp.maximum(a, b), jnp.minimum(a, b)
        j //= 2
    return arr


def _conv_topk_body(x_ref, w_ref, b_ref, out_ref, y_sc):
    n = pl.program_id(1)
    yb = jnp.dot(w_ref[...], x_ref[0], preferred_element_type=jnp.float32)
    y_sc[pl.ds(n * NBLK, NBLK), :] = yb.T + b_ref[...]

    @pl.when(n == NBLKS - 1)
    def _():
        grp = [y_sc[a * 128:(a + 1) * 128, :] for a in range(NT)]
        grp = _bitonic_sort_desc(grp)

        w = 64
        packed = []
        for a in range(NT):
            lo_a, hi_a = grp[a][:w, :], grp[NT - 1 - a][w:, :]
            packed.append(jnp.concatenate(
                [jnp.maximum(lo_a, hi_a), jnp.minimum(lo_a, hi_a)], axis=0))
        packed = _bitonic_clean_desc(packed)

        while w > 1:
            h = w // 2
            nxt = []
            for a in range(NT):
                am, bm = packed[a][:h, :], packed[NT - 1 - a][h:w, :]
                an, bn = packed[a][w:w + h, :], packed[NT - 1 - a][w + h:, :]
                nxt.append(jnp.concatenate(
                    [jnp.maximum(am, bm), jnp.minimum(an, bn)], axis=0))
            packed = _bitonic_clean_desc(nxt)
            w = h

        for a in range(NT):
            out_ref[0, a:a + 1, :] = packed[a][0:1, :]
            out_ref[0, R + a:R + a + 1, :] = packed[a][1:2, :]


def _mlp_body(f_ref, w1_ref, b1_ref, w2_ref, b2_ref, w3_ref, b3_ref, out_ref):
    h = jax.nn.sigmoid(
        jnp.dot(f_ref[...], w1_ref[...], preferred_element_type=jnp.float32)
        + b1_ref[...]
    )
    h = jax.nn.sigmoid(
        jnp.dot(h, w2_ref[...], preferred_element_type=jnp.float32) + b2_ref[...]
    )
    out_ref[...] = (
        jnp.dot(h, w3_ref[...], preferred_element_type=jnp.float32) + b3_ref[...]
    )


@jax.jit
def kernel(x, conv_w, conv_b, w1, b1, w2, b2, w3, b3):
    topk_t = pl.pallas_call(
        _conv_topk_body,
        grid=(B, NBLKS),
        in_specs=[
            pl.BlockSpec((1, C, NBLK), lambda b, n: (b, 0, n)),
            pl.BlockSpec((J, C), lambda b, n: (0, 0)),
            pl.BlockSpec((1, J), lambda b, n: (0, 0)),
        ],
        out_specs=pl.BlockSpec((1, 2 * R, J), lambda b, n: (b, 0, 0)),
        out_shape=jax.ShapeDtypeStruct((B, 2 * R, J), jnp.float32),
        scratch_shapes=[pltpu.VMEM((N, J), jnp.float32)],
        compiler_params=pltpu.CompilerParams(
            dimension_semantics=("arbitrary", "arbitrary"),
        ),
    )(x, conv_w, conv_b.reshape(1, J))

    flat = topk_t.transpose(0, 2, 1).reshape(B, 2 * R * J)
    logits = pl.pallas_call(
        _mlp_body,
        in_specs=[pl.BlockSpec(a.shape, lambda: (0,) * a.ndim) for a in
                  (flat, w1, b1.reshape(1, -1), w2, b2.reshape(1, -1),
                   w3, b3.reshape(1, -1))],
        out_specs=pl.BlockSpec((B, 2), lambda: (0, 0)),
        out_shape=jax.ShapeDtypeStruct((B, 2), jnp.float32),
    )(flat, w1, b1.reshape(1, -1), w2, b2.reshape(1, -1), w3, b3.reshape(1, -1))
    return logits


# speedup vs baseline: 3.0076x; 1.4386x over previous
import jax
import jax.numpy as jnp
from jax import lax
from jax.experimental import pallas as pl
from jax.experimental.pallas import tpu as pltpu

B, C, N = 8, 1024, 4096
J, R = 64, 32
NBLK = 512
NBLKS = N // NBLK
NT = 32
BP = B // 2


def _bitonic_sort_desc(arr):
    n = len(arr)
    k = 2
    while k <= n:
        j = k // 2
        while j >= 1:
            for i in range(n):
                p = i ^ j
                if p > i:
                    a, b = arr[i], arr[p]
                    hi, lo = jnp.maximum(a, b), jnp.minimum(a, b)
                    if (i & k) == 0:
                        arr[i], arr[p] = hi, lo
                    else:
                        arr[i], arr[p] = lo, hi
            j //= 2
        k *= 2
    return arr


def _bitonic_clean_desc(arr):
    n = len(arr)
    j = n // 2
    while j >= 1:
        for i in range(n):
            p = i ^ j
            if p > i:
                a, b = arr[i], arr[p]
                arr[i], arr[p] = jnp.maximum(a, b), jnp.minimum(a, b)
        j //= 2
    return arr


def _conv_topk_body(x_ref, w_ref, b_ref, out_ref, y_sc):
    n = pl.program_id(1)
    yb0 = jnp.dot(w_ref[...], x_ref[0, 0], preferred_element_type=jnp.float32)
    yb1 = jnp.dot(w_ref[...], x_ref[0, 1], preferred_element_type=jnp.float32)
    yb = jnp.concatenate([yb0, yb1], axis=0)
    y_sc[pl.ds(n * NBLK, NBLK), :] = yb.T + b_ref[...]

    @pl.when(n == NBLKS - 1)
    def _():
        grp = [y_sc[a * 128:(a + 1) * 128, :] for a in range(NT)]
        grp = _bitonic_sort_desc(grp)

        w = 64
        packed = []
        for a in range(NT):
            lo_a, hi_a = grp[a][:w, :], grp[NT - 1 - a][w:, :]
            packed.append(jnp.concatenate(
                [jnp.maximum(lo_a, hi_a), jnp.minimum(lo_a, hi_a)], axis=0))
        packed = _bitonic_clean_desc(packed)

        while w > 1:
            h = w // 2
            nxt = []
            for a in range(NT):
                am, bm = packed[a][:h, :], packed[NT - 1 - a][h:w, :]
                an, bn = packed[a][w:w + h, :], packed[NT - 1 - a][w + h:, :]
                nxt.append(jnp.concatenate(
                    [jnp.maximum(am, bm), jnp.minimum(an, bn)], axis=0))
            packed = _bitonic_clean_desc(nxt)
            w = h

        for a in range(NT):
            out_ref[0, a:a + 1, :] = packed[a][0:1, :]
            out_ref[0, R + a:R + a + 1, :] = packed[a][1:2, :]


def _mlp_body(f_ref, w1_ref, b1_ref, w2_ref, b2_ref, w3_ref, b3_ref, out_ref):
    h = jax.nn.sigmoid(
        jnp.dot(f_ref[...], w1_ref[...], preferred_element_type=jnp.float32)
        + b1_ref[...]
    )
    h = jax.nn.sigmoid(
        jnp.dot(h, w2_ref[...], preferred_element_type=jnp.float32) + b2_ref[...]
    )
    out_ref[...] = (
        jnp.dot(h, w3_ref[...], preferred_element_type=jnp.float32) + b3_ref[...]
    )


@jax.jit
def kernel(x, conv_w, conv_b, w1, b1, w2, b2, w3, b3):
    bb = jnp.concatenate([conv_b, conv_b]).reshape(1, 2 * J)
    topk_t = pl.pallas_call(
        _conv_topk_body,
        grid=(BP, NBLKS),
        in_specs=[
            pl.BlockSpec((1, 2, C, NBLK), lambda p, n: (p, 0, 0, n)),
            pl.BlockSpec((J, C), lambda p, n: (0, 0)),
            pl.BlockSpec((1, 2 * J), lambda p, n: (0, 0)),
        ],
        out_specs=pl.BlockSpec((1, 2 * R, 2 * J), lambda p, n: (p, 0, 0)),
        out_shape=jax.ShapeDtypeStruct((BP, 2 * R, 2 * J), jnp.float32),
        scratch_shapes=[pltpu.VMEM((N, 2 * J), jnp.float32)],
        compiler_params=pltpu.CompilerParams(
            dimension_semantics=("arbitrary", "arbitrary"),
        ),
    )(x.reshape(BP, 2, C, N), conv_w, bb)

    flat = (topk_t.reshape(BP, 2 * R, 2, J)
            .transpose(0, 2, 3, 1).reshape(B, 2 * R * J))
    logits = pl.pallas_call(
        _mlp_body,
        in_specs=[pl.BlockSpec(a.shape, lambda: (0,) * a.ndim) for a in
                  (flat, w1, b1.reshape(1, -1), w2, b2.reshape(1, -1),
                   w3, b3.reshape(1, -1))],
        out_specs=pl.BlockSpec((B, 2), lambda: (0, 0)),
        out_shape=jax.ShapeDtypeStruct((B, 2), jnp.float32),
    )(flat, w1, b1.reshape(1, -1), w2, b2.reshape(1, -1), w3, b3.reshape(1, -1))
    return logits


# speedup vs baseline: 3.3422x; 1.1113x over previous
import jax
import jax.numpy as jnp
from jax import lax
from jax.experimental import pallas as pl
from jax.experimental.pallas import tpu as pltpu

B, C, N = 8, 1024, 4096
J, R = 64, 32
NBLK = 1024
NBLKS = N // NBLK
NT = 32
BP = B // 2


def _bitonic_sort_desc(arr):
    n = len(arr)
    k = 2
    while k <= n:
        j = k // 2
        while j >= 1:
            for i in range(n):
                p = i ^ j
                if p > i:
                    a, b = arr[i], arr[p]
                    hi, lo = jnp.maximum(a, b), jnp.minimum(a, b)
                    if (i & k) == 0:
                        arr[i], arr[p] = hi, lo
                    else:
                        arr[i], arr[p] = lo, hi
            j //= 2
        k *= 2
    return arr


def _bitonic_clean_desc(arr):
    n = len(arr)
    j = n // 2
    while j >= 1:
        for i in range(n):
            p = i ^ j
            if p > i:
                a, b = arr[i], arr[p]
                arr[i], arr[p] = jnp.maximum(a, b), jnp.minimum(a, b)
        j //= 2
    return arr


def _conv_topk_body(x_ref, w_ref, b_ref, out_ref, y_sc):
    n = pl.program_id(1)
    yb0 = jnp.dot(w_ref[...], x_ref[0, 0], preferred_element_type=jnp.float32)
    yb1 = jnp.dot(w_ref[...], x_ref[0, 1], preferred_element_type=jnp.float32)
    yb = jnp.concatenate([yb0, yb1], axis=0)
    y_sc[pl.ds(n * NBLK, NBLK), :] = yb.T + b_ref[...]

    @pl.when(n == NBLKS - 1)
    def _():
        grp = [y_sc[a * 128:(a + 1) * 128, :] for a in range(NT)]
        grp = _bitonic_sort_desc(grp)

        w = 64
        packed = []
        for a in range(NT):
            lo_a, hi_a = grp[a][:w, :], grp[NT - 1 - a][w:, :]
            packed.append(jnp.concatenate(
                [jnp.maximum(lo_a, hi_a), jnp.minimum(lo_a, hi_a)], axis=0))
        packed = _bitonic_clean_desc(packed)

        while w > 1:
            h = w // 2
            nxt = []
            for a in range(NT):
                am, bm = packed[a][:h, :], packed[NT - 1 - a][h:w, :]
                an, bn = packed[a][w:w + h, :], packed[NT - 1 - a][w + h:, :]
                nxt.append(jnp.concatenate(
                    [jnp.maximum(am, bm), jnp.minimum(an, bn)], axis=0))
            packed = _bitonic_clean_desc(nxt)
            w = h

        for a in range(NT):
            out_ref[0, a:a + 1, :] = packed[a][0:1, :]
            out_ref[0, R + a:R + a + 1, :] = packed[a][1:2, :]


def _mlp_body(f_ref, w1_ref, b1_ref, w2_ref, b2_ref, w3_ref, b3_ref, out_ref):
    h = jax.nn.sigmoid(
        jnp.dot(f_ref[...], w1_ref[...], preferred_element_type=jnp.float32)
        + b1_ref[...]
    )
    h = jax.nn.sigmoid(
        jnp.dot(h, w2_ref[...], preferred_element_type=jnp.float32) + b2_ref[...]
    )
    out_ref[...] = (
        jnp.dot(h, w3_ref[...], preferred_element_type=jnp.float32) + b3_ref[...]
    )


@jax.jit
def kernel(x, conv_w, conv_b, w1, b1, w2, b2, w3, b3):
    bb = jnp.concatenate([conv_b, conv_b]).reshape(1, 2 * J)
    topk_t = pl.pallas_call(
        _conv_topk_body,
        grid=(BP, NBLKS),
        in_specs=[
            pl.BlockSpec((1, 2, C, NBLK), lambda p, n: (p, 0, 0, n)),
            pl.BlockSpec((J, C), lambda p, n: (0, 0)),
            pl.BlockSpec((1, 2 * J), lambda p, n: (0, 0)),
        ],
        out_specs=pl.BlockSpec((1, 2 * R, 2 * J), lambda p, n: (p, 0, 0)),
        out_shape=jax.ShapeDtypeStruct((BP, 2 * R, 2 * J), jnp.float32),
        scratch_shapes=[pltpu.VMEM((N, 2 * J), jnp.float32)],
        compiler_params=pltpu.CompilerParams(
            dimension_semantics=("arbitrary", "arbitrary"),
        ),
    )(x.reshape(BP, 2, C, N), conv_w, bb)

    flat = (topk_t.reshape(BP, 2 * R, 2, J)
            .transpose(0, 2, 3, 1).reshape(B, 2 * R * J))
    logits = pl.pallas_call(
        _mlp_body,
        in_specs=[pl.BlockSpec(a.shape, lambda: (0,) * a.ndim) for a in
                  (flat, w1, b1.reshape(1, -1), w2, b2.reshape(1, -1),
                   w3, b3.reshape(1, -1))],
        out_specs=pl.BlockSpec((B, 2), lambda: (0, 0)),
        out_shape=jax.ShapeDtypeStruct((B, 2), jnp.float32),
    )(flat, w1, b1.reshape(1, -1), w2, b2.reshape(1, -1), w3, b3.reshape(1, -1))
    return logits


# speedup vs baseline: 3.5856x; 1.0728x over previous
import jax
import jax.numpy as jnp
from jax import lax
from jax.experimental import pallas as pl
from jax.experimental.pallas import tpu as pltpu

B, C, N = 8, 1024, 4096
J, R = 64, 32
NBLK = 2048
NBLKS = N // NBLK
NT = 32
BP = B // 2


def _bitonic_sort_desc(arr):
    n = len(arr)
    k = 2
    while k <= n:
        j = k // 2
        while j >= 1:
            for i in range(n):
                p = i ^ j
                if p > i:
                    a, b = arr[i], arr[p]
                    hi, lo = jnp.maximum(a, b), jnp.minimum(a, b)
                    if (i & k) == 0:
                        arr[i], arr[p] = hi, lo
                    else:
                        arr[i], arr[p] = lo, hi
            j //= 2
        k *= 2
    return arr


def _bitonic_clean_desc(arr):
    n = len(arr)
    j = n // 2
    while j >= 1:
        for i in range(n):
            p = i ^ j
            if p > i:
                a, b = arr[i], arr[p]
                arr[i], arr[p] = jnp.maximum(a, b), jnp.minimum(a, b)
        j //= 2
    return arr


def _conv_topk_body(x_ref, w_ref, b_ref, out_ref, y_sc):
    n = pl.program_id(1)
    yb0 = jnp.dot(w_ref[...], x_ref[0, 0], preferred_element_type=jnp.float32)
    yb1 = jnp.dot(w_ref[...], x_ref[0, 1], preferred_element_type=jnp.float32)
    yb = jnp.concatenate([yb0, yb1], axis=0)
    y_sc[pl.ds(n * NBLK, NBLK), :] = yb.T + b_ref[...]

    @pl.when(n == NBLKS - 1)
    def _():
        grp = [y_sc[a * 128:(a + 1) * 128, :] for a in range(NT)]
        grp = _bitonic_sort_desc(grp)

        w = 64
        packed = []
        for a in range(NT):
            lo_a, hi_a = grp[a][:w, :], grp[NT - 1 - a][w:, :]
            packed.append(jnp.concatenate(
                [jnp.maximum(lo_a, hi_a), jnp.minimum(lo_a, hi_a)], axis=0))
        packed = _bitonic_clean_desc(packed)

        while w > 1:
            h = w // 2
            nxt = []
            for a in range(NT):
                am, bm = packed[a][:h, :], packed[NT - 1 - a][h:w, :]
                an, bn = packed[a][w:w + h, :], packed[NT - 1 - a][w + h:, :]
                nxt.append(jnp.concatenate(
                    [jnp.maximum(am, bm), jnp.minimum(an, bn)], axis=0))
            packed = _bitonic_clean_desc(nxt)
            w = h

        for a in range(NT):
            out_ref[0, a:a + 1, :] = packed[a][0:1, :]
            out_ref[0, R + a:R + a + 1, :] = packed[a][1:2, :]


def _mlp_body(f_ref, w1_ref, b1_ref, w2_ref, b2_ref, w3_ref, b3_ref, out_ref):
    h = jax.nn.sigmoid(
        jnp.dot(f_ref[...], w1_ref[...], preferred_element_type=jnp.float32)
        + b1_ref[...]
    )
    h = jax.nn.sigmoid(
        jnp.dot(h, w2_ref[...], preferred_element_type=jnp.float32) + b2_ref[...]
    )
    out_ref[...] = (
        jnp.dot(h, w3_ref[...], preferred_element_type=jnp.float32) + b3_ref[...]
    )


@jax.jit
def kernel(x, conv_w, conv_b, w1, b1, w2, b2, w3, b3):
    bb = jnp.concatenate([conv_b, conv_b]).reshape(1, 2 * J)
    topk_t = pl.pallas_call(
        _conv_topk_body,
        grid=(BP, NBLKS),
        in_specs=[
            pl.BlockSpec((1, 2, C, NBLK), lambda p, n: (p, 0, 0, n)),
            pl.BlockSpec((J, C), lambda p, n: (0, 0)),
            pl.BlockSpec((1, 2 * J), lambda p, n: (0, 0)),
        ],
        out_specs=pl.BlockSpec((1, 2 * R, 2 * J), lambda p, n: (p, 0, 0)),
        out_shape=jax.ShapeDtypeStruct((BP, 2 * R, 2 * J), jnp.float32),
        scratch_shapes=[pltpu.VMEM((N, 2 * J), jnp.float32)],
        compiler_params=pltpu.CompilerParams(
            dimension_semantics=("arbitrary", "arbitrary"),
        ),
    )(x.reshape(BP, 2, C, N), conv_w, bb)

    flat = (topk_t.reshape(BP, 2 * R, 2, J)
            .transpose(0, 2, 3, 1).reshape(B, 2 * R * J))
    logits = pl.pallas_call(
        _mlp_body,
        in_specs=[pl.BlockSpec(a.shape, lambda: (0,) * a.ndim) for a in
                  (flat, w1, b1.reshape(1, -1), w2, b2.reshape(1, -1),
                   w3, b3.reshape(1, -1))],
        out_specs=pl.BlockSpec((B, 2), lambda: (0, 0)),
        out_shape=jax.ShapeDtypeStruct((B, 2), jnp.float32),
    )(flat, w1, b1.reshape(1, -1), w2, b2.reshape(1, -1), w3, b3.reshape(1, -1))
    return logits
